# baseline (speedup 1.0000x reference)
import jax
import jax.numpy as jnp
from jax import lax
from jax.experimental import pallas as pl
from jax.experimental.pallas import tpu as pltpu
from jax.experimental.pallas import tpu_sc as plsc

N, C, H, W = 4, 96, 224, 224
P = H * W
NIMG = N * C
NWORKERS = 32
IMGS_PER_WORKER = NIMG // NWORKERS
PAIRS_PER_WORKER = IMGS_PER_WORKER // 2
CH = 1792
NCH = P // CH
VECS = CH // 16
LANES = 16


def _prep_body(gx_ref, gy_ref, i0_ref, i1_ref, i2_ref, i3_ref,
               w0_ref, w1_ref, w2_ref, w3_ref):
    gx = gx_ref[...]
    gy = gy_ref[...]
    ix = (gx + 1.0) * (0.5 * (W - 1))
    iy = (gy + 1.0) * (0.5 * (H - 1))
    ix0 = jnp.floor(ix)
    iy0 = jnp.floor(iy)
    wx1 = ix - ix0
    wx0 = 1.0 - wx1
    wy1 = iy - iy0
    wy0 = 1.0 - wy1

    def corner(xc, yc, wgt, i_ref, w_ref):
        valid = ((xc >= 0.0) & (xc <= W - 1.0)
                 & (yc >= 0.0) & (yc <= H - 1.0))
        xi = jnp.clip(xc, 0.0, W - 1.0).astype(jnp.int32)
        yi = jnp.clip(yc, 0.0, H - 1.0).astype(jnp.int32)
        i_ref[...] = yi * W + xi
        w_ref[...] = wgt * valid.astype(jnp.float32)

    corner(ix0, iy0, wx0 * wy0, i0_ref, w0_ref)
    corner(ix0 + 1.0, iy0, wx1 * wy0, i1_ref, w1_ref)
    corner(ix0, iy0 + 1.0, wx0 * wy1, i2_ref, w2_ref)
    corner(ix0 + 1.0, iy0 + 1.0, wx1 * wy1, i3_ref, w3_ref)


def _prep(gx, gy):
    out = jax.ShapeDtypeStruct((N, P), jnp.int32)
    outw = jax.ShapeDtypeStruct((N, P), jnp.float32)
    return pl.pallas_call(
        _prep_body,
        out_shape=[out, out, out, out, outw, outw, outw, outw],
    )(gx, gy)


def _sc_body(inp_ref, i0_ref, i1_ref, i2_ref, i3_ref,
             w0_ref, w1_ref, w2_ref, w3_ref, out_ref,
             img0, img1, ib0, ib1, ib2, ib3, wb0, wb1, wb2, wb3, ob0, ob1):
    wid = lax.axis_index("s") * 2 + lax.axis_index("c")
    n = wid // (NWORKERS // N)

    for j in range(PAIRS_PER_WORKER):
        f0 = wid * IMGS_PER_WORKER + 2 * j
        pltpu.sync_copy(inp_ref.at[f0], img0)
        pltpu.sync_copy(inp_ref.at[f0 + 1], img1)

        def chunk_body(q, _, f0=f0):
            sl = pl.ds(q * CH, CH)
            pltpu.sync_copy(i0_ref.at[n, sl], ib0)
            pltpu.sync_copy(i1_ref.at[n, sl], ib1)
            pltpu.sync_copy(i2_ref.at[n, sl], ib2)
            pltpu.sync_copy(i3_ref.at[n, sl], ib3)
            pltpu.sync_copy(w0_ref.at[n, sl], wb0)
            pltpu.sync_copy(w1_ref.at[n, sl], wb1)
            pltpu.sync_copy(w2_ref.at[n, sl], wb2)
            pltpu.sync_copy(w3_ref.at[n, sl], wb3)

            def vec_body(v, _):
                vs = pl.ds(v * LANES, LANES)
                ii0 = ib0[vs]
                ii1 = ib1[vs]
                ii2 = ib2[vs]
                ii3 = ib3[vs]
                ww0 = wb0[vs]
                ww1 = wb1[vs]
                ww2 = wb2[vs]
                ww3 = wb3[vs]
                for img, ob in ((img0, ob0), (img1, ob1)):
                    acc = (plsc.load_gather(img, [ii0]) * ww0
                           + plsc.load_gather(img, [ii1]) * ww1
                           + plsc.load_gather(img, [ii2]) * ww2
                           + plsc.load_gather(img, [ii3]) * ww3)
                    ob[vs] = acc
                return 0

            lax.fori_loop(0, VECS, vec_body, 0)
            pltpu.sync_copy(ob0, out_ref.at[f0, sl])
            pltpu.sync_copy(ob1, out_ref.at[f0 + 1, sl])
            return 0

        lax.fori_loop(0, NCH, chunk_body, 0)


def _sc_sample(inp_flat, i0, i1, i2, i3, w0, w1, w2, w3):
    mesh = plsc.VectorSubcoreMesh(core_axis_name="c", subcore_axis_name="s")
    fn = pl.kernel(
        _sc_body,
        out_type=jax.ShapeDtypeStruct((NIMG, P), jnp.float32),
        mesh=mesh,
        compiler_params=pltpu.CompilerParams(needs_layout_passes=False),
        scratch_types=[
            pltpu.VMEM((P,), jnp.float32),
            pltpu.VMEM((P,), jnp.float32),
            pltpu.VMEM((CH,), jnp.int32),
            pltpu.VMEM((CH,), jnp.int32),
            pltpu.VMEM((CH,), jnp.int32),
            pltpu.VMEM((CH,), jnp.int32),
            pltpu.VMEM((CH,), jnp.float32),
            pltpu.VMEM((CH,), jnp.float32),
            pltpu.VMEM((CH,), jnp.float32),
            pltpu.VMEM((CH,), jnp.float32),
            pltpu.VMEM((CH,), jnp.float32),
            pltpu.VMEM((CH,), jnp.float32),
        ],
    )
    return fn(inp_flat, i0, i1, i2, i3, w0, w1, w2, w3)


@jax.jit
def _run(input, grid):
    gx = grid[..., 0].reshape(N, P)
    gy = grid[..., 1].reshape(N, P)
    i0, i1, i2, i3, w0, w1, w2, w3 = _prep(gx, gy)
    inp_flat = input.reshape(NIMG, H * W)
    out_flat = _sc_sample(inp_flat, i0, i1, i2, i3, w0, w1, w2, w3)
    return out_flat.reshape(N, C, H, W)


def kernel(input, grid, interpolation_mode, padding_mode, align_corners):
    return _run(input, grid)


# speedup vs baseline: 2.6722x; 2.6722x over previous
import jax
import jax.numpy as jnp
from jax import lax
from jax.experimental import pallas as pl
from jax.experimental.pallas import tpu as pltpu
from jax.experimental.pallas import tpu_sc as plsc

N, C, H, W = 4, 96, 224, 224
P = H * W
NIMG = N * C
NWORKERS = 32
IMGS_PER_WORKER = NIMG // NWORKERS
PAIRS_PER_WORKER = IMGS_PER_WORKER // 2
CH = 896
NCH = P // CH
NGRP = NCH // 2
LANES = 16


def _prep_body(gx_ref, gy_ref, i_ref, w_ref):
    gx = gx_ref[...]
    gy = gy_ref[...]
    ix = (gx + 1.0) * (0.5 * (W - 1))
    iy = (gy + 1.0) * (0.5 * (H - 1))
    ix0 = jnp.floor(ix)
    iy0 = jnp.floor(iy)
    wx1 = ix - ix0
    wx0 = 1.0 - wx1
    wy1 = iy - iy0
    wy0 = 1.0 - wy1

    def corner(c, xc, yc, wgt):
        valid = ((xc >= 0.0) & (xc <= W - 1.0)
                 & (yc >= 0.0) & (yc <= H - 1.0))
        xi = jnp.clip(xc, 0.0, W - 1.0).astype(jnp.int32)
        yi = jnp.clip(yc, 0.0, H - 1.0).astype(jnp.int32)
        i_ref[:, c, :] = yi * W + xi
        w_ref[:, c, :] = wgt * valid.astype(jnp.float32)

    corner(0, ix0, iy0, wx0 * wy0)
    corner(1, ix0 + 1.0, iy0, wx1 * wy0)
    corner(2, ix0, iy0 + 1.0, wx0 * wy1)
    corner(3, ix0 + 1.0, iy0 + 1.0, wx1 * wy1)


def _prep(gx, gy):
    return pl.pallas_call(
        _prep_body,
        out_shape=[jax.ShapeDtypeStruct((N, 4, P), jnp.int32),
                   jax.ShapeDtypeStruct((N, 4, P), jnp.float32)],
    )(gx, gy)


def _sc_body(inp_ref, idx_ref, wgt_ref, out_ref,
             img0, img1, ibuf, wbuf, obuf,
             semi0, semi1, semo0, semo1):
    wid = lax.axis_index("s") * 2 + lax.axis_index("c")
    n = wid // (NWORKERS // N)
    semi = (semi0, semi1)
    semo = (semo0, semo1)

    def in_copies(b, q):
        sl = pl.ds(q * CH, CH)
        return (pltpu.make_async_copy(idx_ref.at[n, :, sl], ibuf.at[b], semi[b]),
                pltpu.make_async_copy(wgt_ref.at[n, :, sl], wbuf.at[b], semi[b]))

    def out_copies(b, q, f0):
        sl = pl.ds(q * CH, CH)
        return (pltpu.make_async_copy(obuf.at[b, 0], out_ref.at[f0, sl], semo[b]),
                pltpu.make_async_copy(obuf.at[b, 1], out_ref.at[f0 + 1, sl], semo[b]))

    def pair_body(p, _):
        f0 = wid * IMGS_PER_WORKER + 2 * p
        pltpu.sync_copy(inp_ref.at[f0], img0)
        pltpu.sync_copy(inp_ref.at[f0 + 1], img1)

        for b in (0, 1):
            for cp in in_copies(b, b):
                cp.start()

        def group_body(g, _):
            for b in (0, 1):
                q = 2 * g + b
                for cp in in_copies(b, q):
                    cp.wait()

                @pl.when(g > 0)
                def _():
                    for cp in out_copies(b, q - 2, f0):
                        cp.wait()

                @plsc.parallel_loop(0, CH, step=LANES, unroll=2)
                def vec_body(i):
                    vs = pl.ds(i, LANES)
                    ii0 = ibuf[b, 0, vs]
                    ii1 = ibuf[b, 1, vs]
                    ii2 = ibuf[b, 2, vs]
                    ii3 = ibuf[b, 3, vs]
                    ww0 = wbuf[b, 0, vs]
                    ww1 = wbuf[b, 1, vs]
                    ww2 = wbuf[b, 2, vs]
                    ww3 = wbuf[b, 3, vs]
                    for s, img in ((0, img0), (1, img1)):
                        acc = (plsc.load_gather(img, [ii0]) * ww0
                               + plsc.load_gather(img, [ii1]) * ww1
                               + plsc.load_gather(img, [ii2]) * ww2
                               + plsc.load_gather(img, [ii3]) * ww3)
                        obuf[b, s, vs] = acc

                for cp in out_copies(b, q, f0):
                    cp.start()

                @pl.when(g < NGRP - 1)
                def _():
                    for cp in in_copies(b, q + 2):
                        cp.start()
            return 0

        lax.fori_loop(0, NGRP, group_body, 0)
        for b in (0, 1):
            for cp in out_copies(b, NCH - 2 + b, f0):
                cp.wait()
        return 0

    lax.fori_loop(0, PAIRS_PER_WORKER, pair_body, 0)


def _sc_sample(inp_flat, idx, wgt):
    mesh = plsc.VectorSubcoreMesh(core_axis_name="c", subcore_axis_name="s")
    fn = pl.kernel(
        _sc_body,
        out_type=jax.ShapeDtypeStruct((NIMG, P), jnp.float32),
        mesh=mesh,
        compiler_params=pltpu.CompilerParams(needs_layout_passes=False),
        scratch_types=[
            pltpu.VMEM((P,), jnp.float32),
            pltpu.VMEM((P,), jnp.float32),
            pltpu.VMEM((2, 4, CH), jnp.int32),
            pltpu.VMEM((2, 4, CH), jnp.float32),
            pltpu.VMEM((2, 2, CH), jnp.float32),
            pltpu.SemaphoreType.DMA,
            pltpu.SemaphoreType.DMA,
            pltpu.SemaphoreType.DMA,
            pltpu.SemaphoreType.DMA,
        ],
    )
    return fn(inp_flat, idx, wgt)


@jax.jit
def _run(input, grid):
    gx = grid[..., 0].reshape(N, P)
    gy = grid[..., 1].reshape(N, P)
    idx, wgt = _prep(gx, gy)
    inp_flat = input.reshape(NIMG, H * W)
    out_flat = _sc_sample(inp_flat, idx, wgt)
    return out_flat.reshape(N, C, H, W)


def kernel(input, grid, interpolation_mode, padding_mode, align_corners):
    return _run(input, grid)


# speedup vs baseline: 2.6919x; 1.0074x over previous
import jax
import jax.numpy as jnp
from jax import lax
from jax.experimental import pallas as pl
from jax.experimental.pallas import tpu as pltpu
from jax.experimental.pallas import tpu_sc as plsc

N, C, H, W = 4, 96, 224, 224
P = H * W
NIMG = N * C
NWORKERS = 32
IMGS_PER_WORKER = NIMG // NWORKERS
PAIRS_PER_WORKER = IMGS_PER_WORKER // 2
CH = 896
NCH = P // CH
NGRP = NCH // 2
LANES = 16


def _prep_body(gx_ref, gy_ref, i_ref, w_ref):
    gx = gx_ref[...]
    gy = gy_ref[...]
    ix = (gx + 1.0) * (0.5 * (W - 1))
    iy = (gy + 1.0) * (0.5 * (H - 1))
    ix0 = jnp.floor(ix)
    iy0 = jnp.floor(iy)
    wx1 = ix - ix0
    wx0 = 1.0 - wx1
    wy1 = iy - iy0
    wy0 = 1.0 - wy1

    def corner(c, xc, yc, wgt):
        valid = ((xc >= 0.0) & (xc <= W - 1.0)
                 & (yc >= 0.0) & (yc <= H - 1.0))
        xi = jnp.clip(xc, 0.0, W - 1.0).astype(jnp.int32)
        yi = jnp.clip(yc, 0.0, H - 1.0).astype(jnp.int32)
        i_ref[:, c, :] = yi * W + xi
        w_ref[:, c, :] = wgt * valid.astype(jnp.float32)

    corner(0, ix0, iy0, wx0 * wy0)
    corner(1, ix0 + 1.0, iy0, wx1 * wy0)
    corner(2, ix0, iy0 + 1.0, wx0 * wy1)
    corner(3, ix0 + 1.0, iy0 + 1.0, wx1 * wy1)


PREP_GRID = 8
PREP_CH = P // PREP_GRID


def _prep(gx, gy):
    in_blk = pl.BlockSpec((N, PREP_CH), lambda i: (0, i))
    out_blk = pl.BlockSpec((N, 4, PREP_CH), lambda i: (0, 0, i))
    return pl.pallas_call(
        _prep_body,
        grid=(PREP_GRID,),
        in_specs=[in_blk, in_blk],
        out_specs=[out_blk, out_blk],
        out_shape=[jax.ShapeDtypeStruct((N, 4, P), jnp.int32),
                   jax.ShapeDtypeStruct((N, 4, P), jnp.float32)],
    )(gx, gy)


def _sc_body(inp_ref, idx_ref, wgt_ref, out_ref,
             img0, img1, ibuf, wbuf, obuf,
             semi0, semi1, semo0, semo1):
    wid = lax.axis_index("s") * 2 + lax.axis_index("c")
    n = wid // (NWORKERS // N)
    semi = (semi0, semi1)
    semo = (semo0, semo1)

    def in_copies(b, q):
        sl = pl.ds(q * CH, CH)
        return (pltpu.make_async_copy(idx_ref.at[n, :, sl], ibuf.at[b], semi[b]),
                pltpu.make_async_copy(wgt_ref.at[n, :, sl], wbuf.at[b], semi[b]))

    def out_copies(b, q, f0):
        sl = pl.ds(q * CH, CH)
        return (pltpu.make_async_copy(obuf.at[b, 0], out_ref.at[f0, sl], semo[b]),
                pltpu.make_async_copy(obuf.at[b, 1], out_ref.at[f0 + 1, sl], semo[b]))

    def pair_body(p, _):
        f0 = wid * IMGS_PER_WORKER + 2 * p
        pltpu.sync_copy(inp_ref.at[f0], img0)
        pltpu.sync_copy(inp_ref.at[f0 + 1], img1)

        for b in (0, 1):
            for cp in in_copies(b, b):
                cp.start()

        def group_body(g, _):
            for b in (0, 1):
                q = 2 * g + b
                for cp in in_copies(b, q):
                    cp.wait()

                @pl.when(g > 0)
                def _():
                    for cp in out_copies(b, q - 2, f0):
                        cp.wait()

                @plsc.parallel_loop(0, CH, step=LANES, unroll=4)
                def vec_body(i):
                    vs = pl.ds(i, LANES)
                    ii0 = ibuf[b, 0, vs]
                    ii1 = ibuf[b, 1, vs]
                    ii2 = ibuf[b, 2, vs]
                    ii3 = ibuf[b, 3, vs]
                    ww0 = wbuf[b, 0, vs]
                    ww1 = wbuf[b, 1, vs]
                    ww2 = wbuf[b, 2, vs]
                    ww3 = wbuf[b, 3, vs]
                    for s, img in ((0, img0), (1, img1)):
                        acc = (plsc.load_gather(img, [ii0]) * ww0
                               + plsc.load_gather(img, [ii1]) * ww1
                               + plsc.load_gather(img, [ii2]) * ww2
                               + plsc.load_gather(img, [ii3]) * ww3)
                        obuf[b, s, vs] = acc

                for cp in out_copies(b, q, f0):
                    cp.start()

                @pl.when(g < NGRP - 1)
                def _():
                    for cp in in_copies(b, q + 2):
                        cp.start()
            return 0

        lax.fori_loop(0, NGRP, group_body, 0)
        for b in (0, 1):
            for cp in out_copies(b, NCH - 2 + b, f0):
                cp.wait()
        return 0

    lax.fori_loop(0, PAIRS_PER_WORKER, pair_body, 0)


def _sc_sample(inp_flat, idx, wgt):
    mesh = plsc.VectorSubcoreMesh(core_axis_name="c", subcore_axis_name="s")
    fn = pl.kernel(
        _sc_body,
        out_type=jax.ShapeDtypeStruct((NIMG, P), jnp.float32),
        mesh=mesh,
        compiler_params=pltpu.CompilerParams(needs_layout_passes=False),
        scratch_types=[
            pltpu.VMEM((P,), jnp.float32),
            pltpu.VMEM((P,), jnp.float32),
            pltpu.VMEM((2, 4, CH), jnp.int32),
            pltpu.VMEM((2, 4, CH), jnp.float32),
            pltpu.VMEM((2, 2, CH), jnp.float32),
            pltpu.SemaphoreType.DMA,
            pltpu.SemaphoreType.DMA,
            pltpu.SemaphoreType.DMA,
            pltpu.SemaphoreType.DMA,
        ],
    )
    return fn(inp_flat, idx, wgt)


@jax.jit
def _run(input, grid):
    gx = grid[..., 0].reshape(N, P)
    gy = grid[..., 1].reshape(N, P)
    idx, wgt = _prep(gx, gy)
    inp_flat = input.reshape(NIMG, H * W)
    out_flat = _sc_sample(inp_flat, idx, wgt)
    return out_flat.reshape(N, C, H, W)


def kernel(input, grid, interpolation_mode, padding_mode, align_corners):
    return _run(input, grid)


# speedup vs baseline: 2.7152x; 1.0086x over previous
import jax
import jax.numpy as jnp
from jax import lax
from jax.experimental import pallas as pl
from jax.experimental.pallas import tpu as pltpu
from jax.experimental.pallas import tpu_sc as plsc

N, C, H, W = 4, 96, 224, 224
P = H * W
NIMG = N * C
NWORKERS = 32
IMGS_PER_WORKER = NIMG // NWORKERS
PAIRS_PER_WORKER = IMGS_PER_WORKER // 2
CH = 896
NCH = P // CH
NGRP = NCH // 2
LANES = 16


def _prep_body(gx_ref, gy_ref, i_ref, w_ref):
    gx = gx_ref[...]
    gy = gy_ref[...]
    ix = (gx + 1.0) * (0.5 * (W - 1))
    iy = (gy + 1.0) * (0.5 * (H - 1))
    ix0 = jnp.floor(ix)
    iy0 = jnp.floor(iy)
    wx1 = ix - ix0
    wx0 = 1.0 - wx1
    wy1 = iy - iy0
    wy0 = 1.0 - wy1

    def corner(c, xc, yc, wgt):
        valid = ((xc >= 0.0) & (xc <= W - 1.0)
                 & (yc >= 0.0) & (yc <= H - 1.0))
        xi = jnp.clip(xc, 0.0, W - 1.0).astype(jnp.int32)
        yi = jnp.clip(yc, 0.0, H - 1.0).astype(jnp.int32)
        i_ref[:, c, :] = yi * W + xi
        w_ref[:, c, :] = wgt * valid.astype(jnp.float32)

    corner(0, ix0, iy0, wx0 * wy0)
    corner(1, ix0 + 1.0, iy0, wx1 * wy0)
    corner(2, ix0, iy0 + 1.0, wx0 * wy1)
    corner(3, ix0 + 1.0, iy0 + 1.0, wx1 * wy1)


PREP_GRID = 8
PREP_CH = P // PREP_GRID


def _prep(gx, gy):
    in_blk = pl.BlockSpec((N, PREP_CH), lambda i: (0, i))
    out_blk = pl.BlockSpec((N, 4, PREP_CH), lambda i: (0, 0, i))
    return pl.pallas_call(
        _prep_body,
        grid=(PREP_GRID,),
        in_specs=[in_blk, in_blk],
        out_specs=[out_blk, out_blk],
        out_shape=[jax.ShapeDtypeStruct((N, 4, P), jnp.int32),
                   jax.ShapeDtypeStruct((N, 4, P), jnp.float32)],
    )(gx, gy)


def _sc_body(inp_ref, idx_ref, wgt_ref, out_ref,
             img0, img1, ibuf, wbuf, obuf,
             semi0, semi1, semo0, semo1):
    wid = lax.axis_index("s") * 2 + lax.axis_index("c")
    n = wid // (NWORKERS // N)
    semi = (semi0, semi1)
    semo = (semo0, semo1)

    def in_copies(b, q):
        sl = pl.ds(q * CH, CH)
        return (pltpu.make_async_copy(idx_ref.at[n, :, sl], ibuf.at[b], semi[b]),
                pltpu.make_async_copy(wgt_ref.at[n, :, sl], wbuf.at[b], semi[b]))

    def out_copies(b, q, f0):
        sl = pl.ds(q * CH, CH)
        return (pltpu.make_async_copy(obuf.at[b, 0], out_ref.at[f0, sl], semo[b]),
                pltpu.make_async_copy(obuf.at[b, 1], out_ref.at[f0 + 1, sl], semo[b]))

    def pair_body(p, _):
        f0 = wid * IMGS_PER_WORKER + 2 * p
        pltpu.sync_copy(inp_ref.at[f0], img0)
        pltpu.sync_copy(inp_ref.at[f0 + 1], img1)

        for b in (0, 1):
            for cp in in_copies(b, b):
                cp.start()

        def group_body(g, _):
            for b in (0, 1):
                q = 2 * g + b
                for cp in in_copies(b, q):
                    cp.wait()

                @pl.when(g > 0)
                def _():
                    for cp in out_copies(b, q - 2, f0):
                        cp.wait()

                @plsc.parallel_loop(0, CH, step=LANES, unroll=4)
                def vec_body(i):
                    vs = pl.ds(i, LANES)
                    ii0 = ibuf[b, 0, vs]
                    ii1 = ibuf[b, 1, vs]
                    ii2 = ibuf[b, 2, vs]
                    ii3 = ibuf[b, 3, vs]
                    ww0 = wbuf[b, 0, vs]
                    ww1 = wbuf[b, 1, vs]
                    ww2 = wbuf[b, 2, vs]
                    ww3 = wbuf[b, 3, vs]
                    for s, img in ((0, img0), (1, img1)):
                        acc = (plsc.load_gather(img, [ii0]) * ww0
                               + plsc.load_gather(img, [ii1]) * ww1
                               + plsc.load_gather(img, [ii2]) * ww2
                               + plsc.load_gather(img, [ii3]) * ww3)
                        obuf[b, s, vs] = acc

                for cp in out_copies(b, q, f0):
                    cp.start()

                @pl.when(g < NGRP - 1)
                def _():
                    for cp in in_copies(b, q + 2):
                        cp.start()
            return 0

        lax.fori_loop(0, NGRP, group_body, 0)
        for b in (0, 1):
            for cp in out_copies(b, NCH - 2 + b, f0):
                cp.wait()
        return 0

    lax.fori_loop(0, PAIRS_PER_WORKER, pair_body, 0)


def _sc_sample(inp_flat, idx, wgt):
    mesh = plsc.VectorSubcoreMesh(core_axis_name="c", subcore_axis_name="s")
    fn = pl.kernel(
        _sc_body,
        out_type=jax.ShapeDtypeStruct((NIMG, P), jnp.float32),
        mesh=mesh,
        compiler_params=pltpu.CompilerParams(needs_layout_passes=False),
        scratch_types=[
            pltpu.VMEM((P,), jnp.float32),
            pltpu.VMEM((P,), jnp.float32),
            pltpu.VMEM((2, 4, CH), jnp.int32),
            pltpu.VMEM((2, 4, CH), jnp.float32),
            pltpu.VMEM((2, 2, CH), jnp.float32),
            pltpu.SemaphoreType.DMA,
            pltpu.SemaphoreType.DMA,
            pltpu.SemaphoreType.DMA,
            pltpu.SemaphoreType.DMA,
        ],
    )
    return fn(inp_flat, idx, wgt)


@jax.jit
def _run(input, grid):
    gxy = jnp.moveaxis(grid.reshape(N, P, 2), 2, 1)
    gx = gxy[:, 0]
    gy = gxy[:, 1]
    idx, wgt = _prep(gx, gy)
    inp_flat = input.reshape(NIMG, H * W)
    out_flat = _sc_sample(inp_flat, idx, wgt)
    return out_flat.reshape(N, C, H, W)


def kernel(input, grid, interpolation_mode, padding_mode, align_corners):
    return _run(input, grid)
